# baseline (speedup 1.0000x reference)
import functools

import jax
import jax.numpy as jnp
from jax import lax
from jax.experimental import pallas as pl
from jax.experimental.pallas import tpu as pltpu
from jax.experimental.pallas import tpu_sc as plsc

_B = 512
_G = 28
_NB = 2
_CL = 80
_CH = _NB * 5 + _CL
_T = 16384
_ROWS = _B * _G * _G
_NGRAN = _ROWS * _CH // 16

_NW = 32
_TPW = _T // _NW
_GCHUNK = 128
_NCHUNK = _TPW // _GCHUNK
_KG = 7


def _ssqrt(v):
    a = jnp.abs(v) + 1e-6
    bits = plsc.bitcast(a, jnp.int32)
    bits = jnp.int32(0x5F3759DF) - lax.shift_right_arithmetic(bits, 1)
    y = plsc.bitcast(bits, jnp.float32)
    y = y * (1.5 - 0.5 * a * y * y)
    y = y * (1.5 - 0.5 * a * y * y)
    y = y * (1.5 - 0.5 * a * y * y)
    return jnp.sign(v) * (a * y)


_mesh = plsc.VectorSubcoreMesh(core_axis_name="c", subcore_axis_name="s")


@functools.partial(
    pl.kernel,
    mesh=_mesh,
    compiler_params=pltpu.CompilerParams(
        needs_layout_passes=False, use_tc_tiling_on_sc=False
    ),
    out_type=jax.ShapeDtypeStruct((_NW, 16), jnp.float32),
    scratch_types=[
        pltpu.VMEM((_TPW, 8), jnp.float32),
        pltpu.VMEM((_NCHUNK, _KG, _GCHUNK), jnp.int32),
        pltpu.VMEM((_TPW,), jnp.int32),
        pltpu.VMEM((_KG, _TPW, 16), jnp.float32),
        pltpu.VMEM((16,), jnp.float32),
        pltpu.SemaphoreType.DMA,
    ],
)
def _sc_loss(gran_hbm, tgt_hbm, part_hbm, tgt_v, idx_v, rvec_v, rows_v, acc_v, sem):
    wid = lax.axis_index("s") * 2 + lax.axis_index("c")
    base = wid * _TPW
    pltpu.sync_copy(tgt_hbm.at[pl.ds(base, _TPW)], tgt_v)

    lanes = lax.iota(jnp.int32, 16)

    def colv(c):
        return jnp.full((16,), c, jnp.int32)

    def tcol(rid, c):
        return plsc.load_gather(tgt_v, [rid, colv(c)])

    def idx_body(i, carry):
        rid = i * 16 + lanes
        bid = tcol(rid, 7).astype(jnp.int32)
        gx = tcol(rid, 4).astype(jnp.int32)
        gy = tcol(rid, 5).astype(jnp.int32)
        r = bid * (_G * _G) + gx * _G + gy
        rvec_v[pl.ds(i * 16, 16)] = r
        g0 = lax.shift_right_logical(r * _CH, 1 + 1 + 1 + 1)
        for k in range(_KG):
            idx_v[i // 8, k, pl.ds((i % 8) * 16, 16)] = g0 + k
        return carry

    lax.fori_loop(0, _TPW // 16, idx_body, 0)

    copies = [
        pltpu.async_copy(
            gran_hbm.at[idx_v.at[j, k]],
            rows_v.at[k, pl.ds(j * _GCHUNK, _GCHUNK)],
            sem,
        )
        for j in range(_NCHUNK)
        for k in range(_KG)
    ]
    for c in copies:
        c.wait()

    def body(i, acc):
        rid = i * 16 + lanes
        r = rvec_v[pl.ds(i * 16, 16)]
        woff = r * _CH - lax.shift_left(
            lax.shift_right_logical(r * _CH, 4), 4
        )

        def chan(c):
            w = woff + c
            k = lax.shift_right_logical(w, 4)
            lane = w - lax.shift_left(k, 4)
            return plsc.load_gather(rows_v, [k, rid, lane])

        xt = tcol(rid, 0)
        yt = tcol(rid, 1)
        wt = tcol(rid, 2)
        ht = tcol(rid, 3)
        cls = tcol(rid, 6).astype(jnp.int32)

        tt = yt - 3.5 * ht
        bt = yt + 3.5 * ht
        lt = xt - 3.5 * wt
        rt = xt + 3.5 * wt
        at = wt * ht * 49.0

        ious = []
        boxes = []
        for nb in range(_NB):
            xg = chan(colv(nb * 5 + 0))
            yg = chan(colv(nb * 5 + 1))
            wg = chan(colv(nb * 5 + 2))
            hg = chan(colv(nb * 5 + 3))
            cg = chan(colv(nb * 5 + 4))
            tg = yg - 3.5 * hg
            bg = yg + 3.5 * hg
            lg = xg - 3.5 * wg
            rg = xg + 3.5 * wg
            wi = jnp.maximum(jnp.minimum(rg, rt) - jnp.maximum(lg, lt), 0.0)
            hi = jnp.maximum(jnp.minimum(bg, bt) - jnp.maximum(tg, tt), 0.0)
            ai = wi * hi
            ag = wg * hg * 49.0
            tot = at + ag - ai
            safe = jnp.where(tot > 1e-6, tot, 1.0)
            ious.append(jnp.where(tot > 1e-6, ai / safe, 0.0))
            boxes.append((xg, yg, wg, hg, cg))

        sel = ious[1] > ious[0]
        xr, yr, wr, hr, cr = (
            jnp.where(sel, b1, b0) for b0, b1 in zip(boxes[0], boxes[1])
        )

        dx = xt - xr
        dy = yt - yr
        dw = _ssqrt(wt) - _ssqrt(wr)
        dh = _ssqrt(ht) - _ssqrt(hr)
        cm1 = cr - 1.0
        acc = acc + 5.0 * (dx * dx + dy * dy)
        acc = acc + 5.0 * (dw * dw + dh * dh)
        acc = acc + cm1 * cm1 - 0.5 * cr * cr

        crc = chan(cls + 10)
        acc = acc + 1.0 - 2.0 * crc

        def cls_body(c, a):
            v = chan(colv(10) + c)
            return a + v * v

        return lax.fori_loop(0, _CL, cls_body, acc)

    acc = lax.fori_loop(0, _TPW // 16, body, jnp.zeros((16,), jnp.float32))
    acc_v[...] = acc
    pltpu.sync_copy(acc_v, part_hbm.at[wid])


_FROWS = _ROWS * _CH // 128
_BLK = 4480


def _tc_body(x_ref, o_ref):
    step = pl.program_id(0)

    @pl.when(step == 0)
    def _init():
        o_ref[...] = jnp.zeros((1, 1), jnp.float32)

    blk = x_ref[...]
    r = lax.broadcasted_iota(jnp.int32, (_BLK, 128), 0)
    lane = lax.broadcasted_iota(jnp.int32, (_BLK, 128), 1)
    pos = ((step * _BLK + r) * 128 + lane) % _CH
    sq = jnp.where((pos == 4) | (pos == 9), blk, 0.0)
    o_ref[...] = o_ref[...] + (0.5 * jnp.sum(sq * sq)).reshape(1, 1)


_tc_noobj = pl.pallas_call(
    _tc_body,
    grid=(_FROWS // _BLK,),
    in_specs=[pl.BlockSpec((_BLK, 128), lambda i: (i, 0))],
    out_specs=pl.BlockSpec((1, 1), lambda i: (0, 0)),
    out_shape=jax.ShapeDtypeStruct((1, 1), jnp.float32),
)


def kernel(output, target):
    gran = output.reshape(_NGRAN, 16)
    part = _sc_loss(gran, target)
    dense = _tc_noobj(gran.reshape(_FROWS, 128))
    return jnp.sum(part) + dense[0, 0]


# speedup vs baseline: 3.9495x; 3.9495x over previous
import functools

import jax
import jax.numpy as jnp
from jax import lax
from jax.experimental import pallas as pl
from jax.experimental.pallas import tpu as pltpu
from jax.experimental.pallas import tpu_sc as plsc

_B = 512
_G = 28
_NB = 2
_CL = 80
_CH = _NB * 5 + _CL
_CHP = 96
_T = 16384
_BQ = _B // 128
_NGRAN = _G * _G * _CHP * _B // 16

_NW = 32
_TPW = _T // _NW
_GCHUNK = 128
_NCHUNK = _TPW // _GCHUNK
_NSLOT = 12


def _ssqrt(v):
    a = jnp.abs(v) + 1e-6
    bits = plsc.bitcast(a, jnp.int32)
    bits = jnp.int32(0x5F3759DF) - lax.shift_right_arithmetic(bits, 1)
    y = plsc.bitcast(bits, jnp.float32)
    y = y * (1.5 - 0.5 * a * y * y)
    y = y * (1.5 - 0.5 * a * y * y)
    y = y * (1.5 - 0.5 * a * y * y)
    return jnp.sign(v) * (a * y)


_mesh = plsc.VectorSubcoreMesh(core_axis_name="c", subcore_axis_name="s")


@functools.partial(
    pl.kernel,
    mesh=_mesh,
    compiler_params=pltpu.CompilerParams(
        needs_layout_passes=False, use_tc_tiling_on_sc=False
    ),
    out_type=jax.ShapeDtypeStruct((_NW, 16), jnp.float32),
    scratch_types=[
        pltpu.VMEM((_TPW, 8), jnp.float32),
        pltpu.VMEM((_NCHUNK, _NSLOT, _GCHUNK), jnp.int32),
        pltpu.VMEM((_TPW,), jnp.int32),
        pltpu.VMEM((_NSLOT, _TPW, 16), jnp.float32),
        pltpu.VMEM((16,), jnp.float32),
        pltpu.SemaphoreType.DMA,
    ],
)
def _sc_loss(gran_hbm, tgt_hbm, part_hbm, tgt_v, idx_v, lane_v, rows_v, acc_v, sem):
    wid = lax.axis_index("s") * 2 + lax.axis_index("c")
    base = wid * _TPW
    pltpu.sync_copy(tgt_hbm.at[pl.ds(base, _TPW)], tgt_v)

    lanes = lax.iota(jnp.int32, 16)

    def colv(c):
        return jnp.full((16,), c, jnp.int32)

    def tcol(rid, c):
        return plsc.load_gather(tgt_v, [rid, colv(c)])

    def idx_body(i, carry):
        rid = i * 16 + lanes
        bid = tcol(rid, 7).astype(jnp.int32)
        gx = tcol(rid, 4).astype(jnp.int32)
        gy = tcol(rid, 5).astype(jnp.int32)
        cls = tcol(rid, 6).astype(jnp.int32)
        cell = gx * _G + gy
        bhi = lax.shift_right_logical(bid, 4)
        lane_v[pl.ds(i * 16, 16)] = bid - lax.shift_left(bhi, 4)
        gbase = (
            (cell * _BQ + lax.shift_right_logical(bid, 7)) * (_CHP * 8)
            + (bhi - lax.shift_left(lax.shift_right_logical(bid, 7), 3))
        )
        for s in range(10):
            idx_v[i // 8, s, pl.ds((i % 8) * 16, 16)] = gbase + s * 8
        idx_v[i // 8, 10, pl.ds((i % 8) * 16, 16)] = gbase + 90 * 8
        idx_v[i // 8, 11, pl.ds((i % 8) * 16, 16)] = gbase + (10 + cls) * 8
        return carry

    lax.fori_loop(0, _TPW // 16, idx_body, 0)

    copies = [
        pltpu.async_copy(
            gran_hbm.at[idx_v.at[j, s]],
            rows_v.at[s, pl.ds(j * _GCHUNK, _GCHUNK)],
            sem,
        )
        for j in range(_NCHUNK)
        for s in range(_NSLOT)
    ]
    for c in copies:
        c.wait()

    def body(i, acc):
        rid = i * 16 + lanes
        lanev = lane_v[pl.ds(i * 16, 16)]

        def chan(s):
            return plsc.load_gather(rows_v, [colv(s), rid, lanev])

        xt = tcol(rid, 0)
        yt = tcol(rid, 1)
        wt = tcol(rid, 2)
        ht = tcol(rid, 3)

        tt = yt - 3.5 * ht
        bt = yt + 3.5 * ht
        lt = xt - 3.5 * wt
        rt = xt + 3.5 * wt
        at = wt * ht * 49.0

        ious = []
        boxes = []
        for nb in range(_NB):
            xg = chan(nb * 5 + 0)
            yg = chan(nb * 5 + 1)
            wg = chan(nb * 5 + 2)
            hg = chan(nb * 5 + 3)
            cg = chan(nb * 5 + 4)
            tg = yg - 3.5 * hg
            bg = yg + 3.5 * hg
            lg = xg - 3.5 * wg
            rg = xg + 3.5 * wg
            wi = jnp.maximum(jnp.minimum(rg, rt) - jnp.maximum(lg, lt), 0.0)
            hi = jnp.maximum(jnp.minimum(bg, bt) - jnp.maximum(tg, tt), 0.0)
            ai = wi * hi
            ag = wg * hg * 49.0
            tot = at + ag - ai
            safe = jnp.where(tot > 1e-6, tot, 1.0)
            ious.append(jnp.where(tot > 1e-6, ai / safe, 0.0))
            boxes.append((xg, yg, wg, hg, cg))

        sel = ious[1] > ious[0]
        xr, yr, wr, hr, cr = (
            jnp.where(sel, b1, b0) for b0, b1 in zip(boxes[0], boxes[1])
        )

        dx = xt - xr
        dy = yt - yr
        dw = _ssqrt(wt) - _ssqrt(wr)
        dh = _ssqrt(ht) - _ssqrt(hr)
        cm1 = cr - 1.0
        acc = acc + 5.0 * (dx * dx + dy * dy)
        acc = acc + 5.0 * (dw * dw + dh * dh)
        acc = acc + cm1 * cm1 - 0.5 * cr * cr
        acc = acc + chan(10)
        acc = acc + 1.0 - 2.0 * chan(11)
        return acc

    acc = lax.fori_loop(0, _TPW // 16, body, jnp.zeros((16,), jnp.float32))
    acc_v[...] = acc
    pltpu.sync_copy(acc_v, part_hbm.at[wid])


def _tc_body(x_ref, img_ref, o_ref):
    step = pl.program_id(0) * pl.num_programs(1) + pl.program_id(1)

    @pl.when(step == 0)
    def _init():
        o_ref[...] = jnp.zeros((1, 1), jnp.float32)

    blk = x_ref[0]
    img_ref[0, :, 0, 0:_CH, :] = blk
    cls = blk[:, 10:_CH, :]
    u = jnp.sum(cls * cls, axis=1, keepdims=True)
    img_ref[0, :, 0, _CH:_CH + 1, :] = u
    c4 = blk[:, 4:5, :]
    c9 = blk[:, 9:10, :]
    noobj = jnp.sum(c4 * c4) + jnp.sum(c9 * c9)
    o_ref[...] = o_ref[...] + (0.5 * noobj).reshape(1, 1)


_tc_extract = pl.pallas_call(
    _tc_body,
    grid=(_G, _BQ),
    in_specs=[pl.BlockSpec((1, _G, _CH, 128), lambda i, q: (i, 0, 0, q))],
    out_specs=[
        pl.BlockSpec((1, _G, 1, _CHP, 128), lambda i, q: (i, 0, q, 0, 0)),
        pl.BlockSpec((1, 1), lambda i, q: (0, 0)),
    ],
    out_shape=[
        jax.ShapeDtypeStruct((_G, _G, _BQ, _CHP, 128), jnp.float32),
        jax.ShapeDtypeStruct((1, 1), jnp.float32),
    ],
)


def kernel(output, target):
    xt = jnp.transpose(output, (1, 2, 3, 0))
    img, noobj = _tc_extract(xt)
    part = _sc_loss(img.reshape(_NGRAN, 16), target)
    return jnp.sum(part) + noobj[0, 0]
